# baseline (speedup 1.0000x reference)
import functools

import jax
import jax.numpy as jnp
from jax.experimental import pallas as pl
from jax.experimental.pallas import tpu as pltpu

_ALPHA = 0.2
_WDT = jnp.bfloat16


def _node_arrays(h, fh, x, wf_ref, ws_ref, wt_ref):
    n = x.shape[0]
    wh = jnp.dot(x, wf_ref[...], preferred_element_type=jnp.float32)
    s = jnp.dot(x, ws_ref[...], preferred_element_type=jnp.float32)
    t = jnp.dot(x, wt_ref[...], preferred_element_type=jnp.float32)
    t_t = t.T
    ones = jnp.ones((n, 1), jnp.float32)
    whe = jnp.concatenate(
        [jnp.concatenate([wh[:, i * fh:(i + 1) * fh], ones], axis=1)
         for i in range(h)], axis=1)
    c = lambda a: a.astype(_WDT)
    return (c(-s), c(jnp.exp(s)), c(jnp.exp(_ALPHA * s)),
            c(t_t), c(jnp.exp(t_t)), c(jnp.exp(_ALPHA * t_t)), c(whe))


def _pre0_kernel(h, fh, x_ref, wf_ref, ws_ref, wt_ref,
                 ns_ref, p_ref, q_ref, t_t_ref, u_t_ref, v_t_ref, whe_ref):
    ns, p, q, t_t, u_t, v_t, whe = _node_arrays(h, fh, x_ref[...],
                                                wf_ref, ws_ref, wt_ref)
    ns_ref[...], p_ref[...], q_ref[...] = ns, p, q
    t_t_ref[:h, :], u_t_ref[:h, :], v_t_ref[:h, :] = t_t, u_t, v_t
    whe_ref[...] = whe


def _attend(h, fh, bi, i, m, col, t_t_ref, u_t_ref, v_t_ref, whe_ref):
    sl = pl.ds(i * bi, bi)
    outs = []
    for k in range(h):
        ns_c = col(sl, 0, k)
        pos = t_t_ref[k:k + 1, :] > ns_c
        w = jnp.where(pos,
                      col(sl, 1, k) * u_t_ref[k:k + 1, :],
                      col(sl, 2, k) * v_t_ref[k:k + 1, :])
        w = w * m
        nd = jnp.dot(w, whe_ref[:, k * (fh + 1):(k + 1) * (fh + 1)],
                     preferred_element_type=jnp.float32)
        outs.append(nd[:, :fh] / nd[:, fh:fh + 1])
    return outs


def _mega_kernel(layers, bi,
                 adj_ref, ns0_ref, p0_ref, q0_ref, tt0_ref, ut0_ref, vt0_ref,
                 whe0_ref, wf_refs, ws_refs, wt_refs, out_ref,
                 mask_ref, x_ref, npq_ref,
                 t_t_ref, u_t_ref, v_t_ref, whe_ref):
    l = pl.program_id(0)
    i = pl.program_id(1)

    for lc in range(len(layers)):
        h, fh, fin = layers[lc]
        last = lc == len(layers) - 1

        if lc > 0:
            @pl.when(jnp.logical_and(l == lc, i == 0))
            def _(lc=lc, h=h, fh=fh, fin=fin):
                ns, p, q, t_t, u_t, v_t, whe = _node_arrays(
                    h, fh, x_ref[:, :fin],
                    wf_refs[lc - 1], ws_refs[lc - 1], wt_refs[lc - 1])
                npq_ref[:, :h] = ns
                npq_ref[:, 4:4 + h] = p
                npq_ref[:, 8:8 + h] = q
                t_t_ref[:h, :], u_t_ref[:h, :], v_t_ref[:h, :] = t_t, u_t, v_t
                whe_ref[:, :h * (fh + 1)] = whe

        @pl.when(l == lc)
        def _(lc=lc, h=h, fh=fh, last=last):
            sl = pl.ds(i * bi, bi)
            if lc == 0:
                m = adj_ref[...].astype(_WDT)
                mask_ref[sl, :] = m
                refs0 = (ns0_ref, p0_ref, q0_ref)
                col0 = lambda sl, kind, k: refs0[kind][sl, k:k + 1]
                outs = _attend(h, fh, bi, i, m, col0,
                               tt0_ref, ut0_ref, vt0_ref, whe0_ref)
            else:
                m = mask_ref[sl, :]
                col = lambda sl, kind, k: npq_ref[sl, 4 * kind + k:
                                                  4 * kind + k + 1]
                outs = _attend(h, fh, bi, i, m, col,
                               t_t_ref, u_t_ref, v_t_ref, whe_ref)
            if not last:
                o = jnp.concatenate(outs, axis=1)
                x_ref[sl, :h * fh] = jnp.where(o > 0, o, jnp.exp(o) - 1.0)
            else:
                o = outs[0]
                for x in outs[1:]:
                    o = o + x
                o = jnp.maximum(o * (1.0 / h), 0.0)
                part = jnp.sum(o, axis=0, keepdims=True)

                @pl.when(i == 0)
                def _():
                    out_ref[...] = jnp.zeros_like(out_ref)

                out_ref[...] += part


def kernel(node_features, adj_mat,
           W1, a1_src, a1_dst,
           W2, a2_src, a2_dst,
           W3, a3_src, a3_dst,
           W4, a4_src, a4_dst):
    n = node_features.shape[0]
    bi = min(256, n)
    nb = n // bi
    params = ((W1, a1_src, a1_dst), (W2, a2_src, a2_dst),
              (W3, a3_src, a3_dst), (W4, a4_src, a4_dst))
    layers = tuple((w.shape[1], w.shape[2], w.shape[0]) for w, _, _ in params)
    wfs, wss, wts = [], [], []
    for w, a_s, a_d in params:
        fin, h, fh = w.shape
        wf = w.reshape(fin, h * fh)
        wfs.append(wf)
        eye = jnp.eye(h, dtype=w.dtype)
        bd_s = (a_s[:, :, None] * eye[:, None, :]).reshape(h * fh, h)
        bd_d = (a_d[:, :, None] * eye[:, None, :]).reshape(h * fh, h)
        wss.append(wf @ bd_s)
        wts.append(wf @ bd_d)

    h0, fh0, _ = layers[0]
    pre0 = pl.pallas_call(
        functools.partial(_pre0_kernel, h0, fh0),
        out_shape=(
            jax.ShapeDtypeStruct((n, h0), _WDT),
            jax.ShapeDtypeStruct((n, h0), _WDT),
            jax.ShapeDtypeStruct((n, h0), _WDT),
            jax.ShapeDtypeStruct((8, n), _WDT),
            jax.ShapeDtypeStruct((8, n), _WDT),
            jax.ShapeDtypeStruct((8, n), _WDT),
            jax.ShapeDtypeStruct((n, h0 * (fh0 + 1)), _WDT),
        ),
    )
    pre0_out = pre0(node_features, wfs[0], wss[0], wts[0])

    full = lambda a: pl.BlockSpec(a.shape, lambda l, i: (0,) * a.ndim)
    max_whe = max(h * (fh + 1) for h, fh, _ in layers[1:])
    max_xf = max(h * fh for h, fh, _ in layers[:-1])
    fh_last = layers[-1][1]
    out = pl.pallas_call(
        functools.partial(_mega_kernel, layers, bi),
        grid=(len(layers), nb),
        compiler_params=pltpu.CompilerParams(vmem_limit_bytes=67_043_328),
        in_specs=[
            pl.BlockSpec((bi, n), lambda l, i: (jnp.where(l == 0, i, 0), 0)),
            *[full(a) for a in pre0_out],
            [full(w) for w in wfs[1:]],
            [full(w) for w in wss[1:]],
            [full(w) for w in wts[1:]],
        ],
        out_specs=pl.BlockSpec((1, fh_last), lambda l, i: (0, 0)),
        out_shape=jax.ShapeDtypeStruct((1, fh_last), jnp.float32),
        scratch_shapes=[
            pltpu.VMEM((n, n), _WDT),
            pltpu.VMEM((n, max_xf), jnp.float32),
            pltpu.VMEM((n, 12), _WDT),
            pltpu.VMEM((8, n), _WDT),
            pltpu.VMEM((8, n), _WDT),
            pltpu.VMEM((8, n), _WDT),
            pltpu.VMEM((n, max_whe), _WDT),
        ],
    )(adj_mat, *pre0_out, wfs[1:], wss[1:], wts[1:])
    return out.reshape(-1)
